# baseline (speedup 1.0000x reference)
import functools

import jax
import jax.numpy as jnp
from jax import lax
from jax.experimental import pallas as pl
from jax.experimental.pallas import tpu as pltpu
from jax.experimental.pallas import tpu_sc as plsc

T = 16384
H = 2048
E = 16

TILE_T = 512

NUM_WORKERS = 32
PER_W = T // NUM_WORKERS
GROUPS = PER_W // 16


def _gate_body(h_ref, w_ref, o_ref, ot_ref):
    acc = jnp.dot(h_ref[...], w_ref[...],
                  preferred_element_type=jnp.float32)
    acc = jnp.clip(acc, -50.0, 50.0)
    o_ref[...] = acc
    ot_ref[...] = acc.T


@jax.jit
def _gate_logits(hidden_states, gate_wt):
    return pl.pallas_call(
        _gate_body,
        grid=(T // TILE_T,),
        in_specs=[
            pl.BlockSpec((TILE_T, H), lambda i: (i, 0)),
            pl.BlockSpec((H, E), lambda i: (0, 0)),
        ],
        out_specs=[
            pl.BlockSpec((TILE_T, E), lambda i: (i, 0)),
            pl.BlockSpec((E, TILE_T), lambda i: (0, i)),
        ],
        out_shape=[
            jax.ShapeDtypeStruct((T, E), jnp.float32),
            jax.ShapeDtypeStruct((E, T), jnp.float32),
        ],
    )(hidden_states, gate_wt)


def _route_body(lgt_hbm, wt_hbm, it_hbm, lgt_v, wt_v, it_v):
    wid = lax.axis_index("s") * 2 + lax.axis_index("c")
    base = wid * PER_W
    pltpu.sync_copy(lgt_hbm.at[:, pl.ds(base, PER_W)], lgt_v)

    def group(g, carry):
        off = g * 16
        m1 = lgt_v[0, pl.ds(off, 16)]
        i1 = jnp.zeros((16,), jnp.int32)
        m2 = jnp.full((16,), -jnp.inf, jnp.float32)
        i2 = jnp.zeros((16,), jnp.int32)
        for e in range(1, E):
            e_vec = jnp.full((16,), e, jnp.int32)
            v = lgt_v[e, pl.ds(off, 16)]
            gt1 = v > m1
            gt2 = v > m2
            m2 = jnp.where(gt1, m1, jnp.where(gt2, v, m2))
            i2 = jnp.where(gt1, i1, jnp.where(gt2, e_vec, i2))
            m1 = jnp.where(gt1, v, m1)
            i1 = jnp.where(gt1, e_vec, i1)
        e2 = jnp.exp(m2 - m1)
        denom = 1.0 + e2
        wt_v[0, pl.ds(off, 16)] = 1.0 / denom
        wt_v[1, pl.ds(off, 16)] = e2 / denom
        it_v[0, pl.ds(off, 16)] = i1
        it_v[1, pl.ds(off, 16)] = i2
        return carry

    lax.fori_loop(0, GROUPS, group, 0)

    pltpu.sync_copy(wt_v, wt_hbm.at[:, pl.ds(base, PER_W)])
    pltpu.sync_copy(it_v, it_hbm.at[:, pl.ds(base, PER_W)])


@jax.jit
def _route(logits_t):
    mesh = plsc.VectorSubcoreMesh(core_axis_name="c", subcore_axis_name="s")
    f = functools.partial(
        pl.kernel,
        mesh=mesh,
        out_type=(
            jax.ShapeDtypeStruct((2, T), jnp.float32),
            jax.ShapeDtypeStruct((2, T), jnp.int32),
        ),
        scratch_types=[
            pltpu.VMEM((E, PER_W), jnp.float32),
            pltpu.VMEM((2, PER_W), jnp.float32),
            pltpu.VMEM((2, PER_W), jnp.int32),
        ],
    )(_route_body)
    return f(logits_t)


def kernel(hidden_states, gate_w):
    logits, logits_t = _gate_logits(hidden_states, gate_w.T)
    w_t, i_t = _route(logits_t)
    return (logits, w_t.T, i_t.T)


# speedup vs baseline: 1.1071x; 1.1071x over previous
import functools

import jax
import jax.numpy as jnp
from jax import lax
from jax.experimental import pallas as pl
from jax.experimental.pallas import tpu as pltpu
from jax.experimental.pallas import tpu_sc as plsc

T = 16384
H = 2048
E = 16

TILE_T = 1024

NUM_WORKERS = 32
PER_W = T // NUM_WORKERS
GROUPS = PER_W // 16


def _gate_body(h_ref, w_ref, o_ref, ot_ref):
    acc = jnp.dot(h_ref[...], w_ref[...],
                  preferred_element_type=jnp.float32)
    acc = jnp.clip(acc, -50.0, 50.0)
    o_ref[...] = acc
    ot_ref[...] = acc.T


@jax.jit
def _gate_logits(hidden_states, gate_wt):
    return pl.pallas_call(
        _gate_body,
        grid=(T // TILE_T,),
        in_specs=[
            pl.BlockSpec((TILE_T, H), lambda i: (i, 0)),
            pl.BlockSpec((H, E), lambda i: (0, 0)),
        ],
        out_specs=[
            pl.BlockSpec((TILE_T, E), lambda i: (i, 0)),
            pl.BlockSpec((E, TILE_T), lambda i: (0, i)),
        ],
        out_shape=[
            jax.ShapeDtypeStruct((T, E), jnp.float32),
            jax.ShapeDtypeStruct((E, T), jnp.float32),
        ],
    )(hidden_states, gate_wt)


def _route_body(lgt_hbm, wt_hbm, it_hbm, lgt_v, wt_v, it_v):
    wid = lax.axis_index("s") * 2 + lax.axis_index("c")
    base = wid * PER_W
    pltpu.sync_copy(lgt_hbm.at[:, pl.ds(base, PER_W)], lgt_v)

    def group(g, carry):
        off = g * 16
        m1 = lgt_v[0, pl.ds(off, 16)]
        i1 = jnp.zeros((16,), jnp.int32)
        m2 = jnp.full((16,), -jnp.inf, jnp.float32)
        i2 = jnp.zeros((16,), jnp.int32)
        for e in range(1, E):
            e_vec = jnp.full((16,), e, jnp.int32)
            v = lgt_v[e, pl.ds(off, 16)]
            gt1 = v > m1
            gt2 = v > m2
            m2 = jnp.where(gt1, m1, jnp.where(gt2, v, m2))
            i2 = jnp.where(gt1, i1, jnp.where(gt2, e_vec, i2))
            m1 = jnp.where(gt1, v, m1)
            i1 = jnp.where(gt1, e_vec, i1)
        e2 = jnp.exp(m2 - m1)
        denom = 1.0 + e2
        wt_v[0, pl.ds(off, 16)] = 1.0 / denom
        wt_v[1, pl.ds(off, 16)] = e2 / denom
        it_v[0, pl.ds(off, 16)] = i1
        it_v[1, pl.ds(off, 16)] = i2
        return carry

    lax.fori_loop(0, GROUPS, group, 0)

    pltpu.sync_copy(wt_v, wt_hbm.at[:, pl.ds(base, PER_W)])
    pltpu.sync_copy(it_v, it_hbm.at[:, pl.ds(base, PER_W)])


@jax.jit
def _route(logits_t):
    mesh = plsc.VectorSubcoreMesh(core_axis_name="c", subcore_axis_name="s")
    f = functools.partial(
        pl.kernel,
        mesh=mesh,
        out_type=(
            jax.ShapeDtypeStruct((2, T), jnp.float32),
            jax.ShapeDtypeStruct((2, T), jnp.int32),
        ),
        scratch_types=[
            pltpu.VMEM((E, PER_W), jnp.float32),
            pltpu.VMEM((2, PER_W), jnp.float32),
            pltpu.VMEM((2, PER_W), jnp.int32),
        ],
    )(_route_body)
    return f(logits_t)


def kernel(hidden_states, gate_w):
    logits, logits_t = _gate_logits(hidden_states, gate_w.T)
    w_t, i_t = _route(logits_t)
    return (logits, w_t.T, i_t.T)
